# Initial kernel scaffold; baseline (speedup 1.0000x reference)
import jax
import jax.numpy as jnp
from jax.experimental import pallas as pl


def kernel(x, edge_index, batch, W1, b1, g1, be1, W2, b2, g2, be2, W3, b3, g3, be3, fcW, fcb):
    raise NotImplementedError("write your pallas kernel here")


# speedup vs baseline: 8.2402x; 8.2402x over previous
import functools

import jax
import jax.numpy as jnp
from jax import lax
from jax.experimental import pallas as pl
from jax.experimental.pallas import tpu as pltpu
from jax.experimental.pallas import tpu_sc as plsc

_N = 10000
_E = 320000
_FIN = 128
_DIM = 64
_YW = 128
_NG = 64
_NC, _NS = 2, 16
_NW = _NC * _NS
_K = 128
_CH = (_E + _NW * _K - 1) // (_NW * _K)
_EPT = _K * _CH
_EPAD = _NW * _EPT
_AR = 10240
_RPT = _AR // _NS
_RB = 1000
_EPS = 1e-5


@functools.cache
def _sc_kernels():
    mesh = plsc.VectorSubcoreMesh(
        core_axis_name="c", subcore_axis_name="s",
        num_cores=_NC, num_subcores=_NS,
    )

    @functools.partial(
        pl.kernel,
        out_type=jax.ShapeDtypeStruct((_NC, _AR, _YW), jnp.float32),
        mesh=mesh,
        scratch_types=[
            pltpu.VMEM((_K,), jnp.int32),
            pltpu.VMEM((_K,), jnp.int32),
            pltpu.VMEM((_K, _YW), jnp.float32),
            pltpu.VMEM_SHARED((_AR, _YW), jnp.float32),
            pltpu.SemaphoreType.DMA,
        ],
    )
    def _sc_agg(y_hbm, src_hbm, dst_hbm, zero_hbm, out_hbm, src_v, dst_v,
                rows_v, acc_sh, sem):
        c = lax.axis_index("c")
        s = lax.axis_index("s")
        wid = c * _NS + s
        pltpu.sync_copy(zero_hbm, acc_sh.at[pl.ds(s * _RPT, _RPT)])
        plsc.subcore_barrier()

        def body(j, carry):
            base = wid * _EPT + j * _K
            pltpu.sync_copy(src_hbm.at[pl.ds(base, _K)], src_v)
            pltpu.sync_copy(dst_hbm.at[pl.ds(base, _K)], dst_v)
            pltpu.async_copy(y_hbm.at[src_v], rows_v, sem).wait()
            pltpu.sync_copy(rows_v, acc_sh.at[dst_v], add=True)
            return carry

        lax.fori_loop(0, _CH, body, 0)
        plsc.subcore_barrier()
        pltpu.sync_copy(acc_sh.at[pl.ds(s * _RPT, _RPT)],
                        out_hbm.at[c, pl.ds(s * _RPT, _RPT)])

    @functools.partial(
        pl.kernel,
        out_type=jax.ShapeDtypeStruct((_NC, _AR), jnp.float32),
        mesh=mesh,
        scratch_types=[
            pltpu.VMEM((_K,), jnp.int32),
            pltpu.VMEM((_K,), jnp.float32),
            pltpu.VMEM_SHARED((_AR,), jnp.float32),
        ],
    )
    def _sc_deg(dst_hbm, one_hbm, zero_hbm, out_hbm, dst_v, ones_v, acc_sh):
        c = lax.axis_index("c")
        s = lax.axis_index("s")
        wid = c * _NS + s
        pltpu.sync_copy(zero_hbm, acc_sh.at[pl.ds(s * _RPT, _RPT)])
        pltpu.sync_copy(one_hbm, ones_v)
        plsc.subcore_barrier()

        def body(j, carry):
            base = wid * _EPT + j * _K
            pltpu.sync_copy(dst_hbm.at[pl.ds(base, _K)], dst_v)
            pltpu.sync_copy(ones_v, acc_sh.at[dst_v], add=True)
            return carry

        lax.fori_loop(0, _CH, body, 0)
        plsc.subcore_barrier()
        pltpu.sync_copy(acc_sh.at[pl.ds(s * _RPT, _RPT)],
                        out_hbm.at[c, pl.ds(s * _RPT, _RPT)])

    return _sc_agg, _sc_deg


def _mm1_body(x_ref, w_ref, degp_ref, y_ref, dinv_ref):
    d = degp_ref[0] + degp_ref[1] + 1.0
    dinv = lax.rsqrt(d)
    dinv_ref[...] = dinv
    y = dinv * jnp.dot(x_ref[...], w_ref[...],
                       preferred_element_type=jnp.float32)
    y_ref[...] = jnp.concatenate(
        [y, jnp.zeros((_RB, _YW - _DIM), jnp.float32)], axis=1)


def _zstat_body(aggp_ref, y_ref, dinv_ref, b_ref, z_ref, st_ref):
    i = pl.program_id(0)
    a = aggp_ref[0] + aggp_ref[1] + y_ref[...]
    z = dinv_ref[...] * a[:, :_DIM] + b_ref[...]
    z_ref[...] = z
    s = jnp.sum(z, axis=0, keepdims=True)
    ss = jnp.sum(z * z, axis=0, keepdims=True)
    st = jnp.concatenate([s, ss], axis=0)

    @pl.when(i == 0)
    def _():
        st_ref[...] = st

    @pl.when(i > 0)
    def _():
        st_ref[...] = st_ref[...] + st


def _bnmm_body(z_ref, st_ref, g_ref, be_ref, w_ref, dinv_ref, y_ref):
    mu = st_ref[0:1] / _N
    var = st_ref[1:2] / _N - mu * mu
    rstd = lax.rsqrt(var + _EPS)
    h = jnp.maximum((z_ref[...] - mu) * (rstd * g_ref[...]) + be_ref[...], 0.0)
    y = dinv_ref[...] * jnp.dot(h, w_ref[...],
                                preferred_element_type=jnp.float32)
    y_ref[...] = jnp.concatenate(
        [y, jnp.zeros((_RB, _YW - _DIM), jnp.float32)], axis=1)


def _final_body(z_ref, st_ref, g_ref, be_ref, batch_ref, fcw_ref, fcb_ref,
                h_ref, ge_ref, out_ref, m_acc):
    i = pl.program_id(0)
    mu = st_ref[0:1] / _N
    var = st_ref[1:2] / _N - mu * mu
    rstd = lax.rsqrt(var + _EPS)
    h = jnp.maximum((z_ref[...] - mu) * (rstd * g_ref[...]) + be_ref[...], 0.0)
    h_ref[...] = h
    bb = batch_ref[...]

    @pl.when(i == 0)
    def _():
        m_acc[...] = jnp.full((_NG, _DIM), -jnp.inf, jnp.float32)

    def upd(g, carry):
        row = jnp.max(jnp.where(bb == g, h, -jnp.inf), axis=0, keepdims=True)
        m_acc[pl.ds(g, 1), :] = jnp.maximum(m_acc[pl.ds(g, 1), :], row)
        return carry

    lax.fori_loop(0, _NG, upd, 0)
    ge_ref[...] = m_acc[...]
    out_ref[...] = jnp.dot(m_acc[...], fcw_ref[...],
                           preferred_element_type=jnp.float32) + fcb_ref[...]


_GRID = _N // _RB


def _mm1(x, W1, degp3):
    return pl.pallas_call(
        _mm1_body,
        grid=(_GRID,),
        in_specs=[
            pl.BlockSpec((_RB, _FIN), lambda i: (i, 0)),
            pl.BlockSpec((_FIN, _DIM), lambda i: (0, 0)),
            pl.BlockSpec((_NC, _RB, 1), lambda i: (0, i, 0)),
        ],
        out_specs=[
            pl.BlockSpec((_RB, _YW), lambda i: (i, 0)),
            pl.BlockSpec((_RB, 1), lambda i: (i, 0)),
        ],
        out_shape=[
            jax.ShapeDtypeStruct((_N, _YW), jnp.float32),
            jax.ShapeDtypeStruct((_N, 1), jnp.float32),
        ],
    )(x, W1, degp3)


def _zstat(aggp, y, dinv, b):
    return pl.pallas_call(
        _zstat_body,
        grid=(_GRID,),
        in_specs=[
            pl.BlockSpec((_NC, _RB, _YW), lambda i: (0, i, 0)),
            pl.BlockSpec((_RB, _YW), lambda i: (i, 0)),
            pl.BlockSpec((_RB, 1), lambda i: (i, 0)),
            pl.BlockSpec((1, _DIM), lambda i: (0, 0)),
        ],
        out_specs=[
            pl.BlockSpec((_RB, _DIM), lambda i: (i, 0)),
            pl.BlockSpec((2, _DIM), lambda i: (0, 0)),
        ],
        out_shape=[
            jax.ShapeDtypeStruct((_N, _DIM), jnp.float32),
            jax.ShapeDtypeStruct((2, _DIM), jnp.float32),
        ],
    )(aggp, y, dinv, b)


def _bnmm(z, st, g, be, W, dinv):
    return pl.pallas_call(
        _bnmm_body,
        grid=(_GRID,),
        in_specs=[
            pl.BlockSpec((_RB, _DIM), lambda i: (i, 0)),
            pl.BlockSpec((2, _DIM), lambda i: (0, 0)),
            pl.BlockSpec((1, _DIM), lambda i: (0, 0)),
            pl.BlockSpec((1, _DIM), lambda i: (0, 0)),
            pl.BlockSpec((_DIM, _DIM), lambda i: (0, 0)),
            pl.BlockSpec((_RB, 1), lambda i: (i, 0)),
        ],
        out_specs=pl.BlockSpec((_RB, _YW), lambda i: (i, 0)),
        out_shape=jax.ShapeDtypeStruct((_N, _YW), jnp.float32),
    )(z, st, g, be, W, dinv)


def _final(z, st, g, be, batch2, fcW, fcb):
    return pl.pallas_call(
        _final_body,
        grid=(_GRID,),
        in_specs=[
            pl.BlockSpec((_RB, _DIM), lambda i: (i, 0)),
            pl.BlockSpec((2, _DIM), lambda i: (0, 0)),
            pl.BlockSpec((1, _DIM), lambda i: (0, 0)),
            pl.BlockSpec((1, _DIM), lambda i: (0, 0)),
            pl.BlockSpec((_RB, 1), lambda i: (i, 0)),
            pl.BlockSpec((_DIM, 2), lambda i: (0, 0)),
            pl.BlockSpec((1, 2), lambda i: (0, 0)),
        ],
        out_specs=[
            pl.BlockSpec((_RB, _DIM), lambda i: (i, 0)),
            pl.BlockSpec((_NG, _DIM), lambda i: (0, 0)),
            pl.BlockSpec((_NG, 2), lambda i: (0, 0)),
        ],
        out_shape=[
            jax.ShapeDtypeStruct((_N, _DIM), jnp.float32),
            jax.ShapeDtypeStruct((_NG, _DIM), jnp.float32),
            jax.ShapeDtypeStruct((_NG, 2), jnp.float32),
        ],
        scratch_shapes=[pltpu.VMEM((_NG, _DIM), jnp.float32)],
    )(z, st, g, be, batch2, fcW, fcb)


def kernel(x, edge_index, batch, W1, b1, g1, be1, W2, b2, g2, be2,
           W3, b3, g3, be3, fcW, fcb):
    src = edge_index[0]
    dst = edge_index[1]
    pad = _EPAD - _E
    srcp = jnp.concatenate([src, jnp.zeros((pad,), jnp.int32)])
    dstp = jnp.concatenate([dst, jnp.full((pad,), _N, jnp.int32)])
    zerosY = jnp.zeros((_RPT, _YW), jnp.float32)
    zeros1 = jnp.zeros((_RPT,), jnp.float32)
    ones1 = jnp.ones((_K,), jnp.float32)

    sc_agg, sc_deg = _sc_kernels()
    degp = sc_deg(dstp, ones1, zeros1)
    degp3 = degp.reshape(_NC, _AR, 1)
    y1, dinv = _mm1(x, W1, degp3)
    agg1 = sc_agg(y1, srcp, dstp, zerosY)
    z1, st1 = _zstat(agg1, y1, dinv, b1.reshape(1, -1))
    y2 = _bnmm(z1, st1, g1.reshape(1, -1), be1.reshape(1, -1), W2, dinv)
    agg2 = sc_agg(y2, srcp, dstp, zerosY)
    z2, st2 = _zstat(agg2, y2, dinv, b2.reshape(1, -1))
    y3 = _bnmm(z2, st2, g2.reshape(1, -1), be2.reshape(1, -1), W3, dinv)
    agg3 = sc_agg(y3, srcp, dstp, zerosY)
    z3, st3 = _zstat(agg3, y3, dinv, b3.reshape(1, -1))
    node, ge, out = _final(z3, st3, g3.reshape(1, -1), be3.reshape(1, -1),
                           batch.reshape(-1, 1), fcW, fcb.reshape(1, -1))
    return (node, ge, out)
